# baseline (speedup 1.0000x reference)
import functools

import jax
import jax.numpy as jnp
from jax import lax
from jax.experimental import pallas as pl
from jax.experimental.pallas import tpu as pltpu
from jax.experimental.pallas import tpu_sc as plsc

_L = 64
_HALF_LOG_2PI = 0.9189385332046727
_KC = 512
_NW = 32
_LANES = 16


def _prep_body(u_ref, w_ref, Wmu_ref, bmu_ref, Wlv_ref, blv_ref,
               c0_ref, c1_ref, c2_ref, m_ref):
    mean = jnp.dot(u_ref[...], Wmu_ref[...],
                   preferred_element_type=jnp.float32) + bmu_ref[...]
    lv = jnp.dot(u_ref[...], Wlv_ref[...],
                 preferred_element_type=jnp.float32) + blv_ref[...]
    wv = w_ref[...]
    wmax = jnp.max(wv)
    logw = wv - (wmax + jnp.log(jnp.sum(jnp.exp(wv - wmax))))
    t = logw - 0.5 * lv - _HALF_LOG_2PI
    m_l = jnp.max(t, axis=0, keepdims=True)
    p = 0.5 * jnp.exp(-lv)
    c0_ref[...] = (t - m_l) - p * mean * mean
    c1_ref[...] = 2.0 * p * mean
    c2_ref[...] = -p
    m_ref[...] = m_l


def _finish_body(s_ref, m_ref, out_ref):
    out_ref[...] = m_ref[...] + jnp.log(s_ref[...])


def _sc_accumulate(z, c0, c1, c2):
    B, L = z.shape
    K = c0.shape[0]
    rows = B // _NW
    mesh = plsc.VectorSubcoreMesh(core_axis_name="c", subcore_axis_name="s")

    @functools.partial(
        pl.kernel, mesh=mesh,
        out_type=jax.ShapeDtypeStruct((B, L), jnp.float32),
        scratch_types=[
            pltpu.VMEM((rows, L), jnp.float32),
            pltpu.VMEM((rows, L), jnp.float32),
            pltpu.VMEM((_KC, L), jnp.float32),
            pltpu.VMEM((_KC, L), jnp.float32),
            pltpu.VMEM((_KC, L), jnp.float32),
        ],
    )
    def sc_kernel(z_hbm, c0_hbm, c1_hbm, c2_hbm, s_hbm,
                  zv, sv, b0, b1, b2):
        wid = lax.axis_index("s") * 2 + lax.axis_index("c")
        base = wid * rows
        pltpu.sync_copy(z_hbm.at[pl.ds(base, rows)], zv)
        nchunks = K // _KC
        for c in range(nchunks):
            pltpu.sync_copy(c0_hbm.at[pl.ds(c * _KC, _KC)], b0)
            pltpu.sync_copy(c1_hbm.at[pl.ds(c * _KC, _KC)], b1)
            pltpu.sync_copy(c2_hbm.at[pl.ds(c * _KC, _KC)], b2)
            for g in range(L // _LANES):
                sl = pl.ds(g * _LANES, _LANES)
                for rr in range(rows // _LANES):
                    zr = [zv[rr * _LANES + r, sl] for r in range(_LANES)]

                    def body(k, acc):
                        v0 = b0[k, sl]
                        v1 = b1[k, sl]
                        v2 = b2[k, sl]
                        return tuple(
                            acc[r] + jnp.exp((v2 * zr[r] + v1) * zr[r] + v0)
                            for r in range(_LANES))

                    acc = lax.fori_loop(
                        0, _KC, body,
                        tuple(jnp.zeros((_LANES,), jnp.float32)
                              for _ in range(_LANES)))
                    for r in range(_LANES):
                        row = rr * _LANES + r
                        if c == 0:
                            sv[row, sl] = acc[r]
                        else:
                            sv[row, sl] = sv[row, sl] + acc[r]
        pltpu.sync_copy(sv, s_hbm.at[pl.ds(base, rows)])

    return sc_kernel(z, c0, c1, c2)


@jax.jit
def kernel(z, u, w, W_mu, b_mu, W_lv, b_lv):
    B, L = z.shape
    K = u.shape[0]
    c0, c1, c2, m_l = pl.pallas_call(
        _prep_body,
        out_shape=[
            jax.ShapeDtypeStruct((K, L), jnp.float32),
            jax.ShapeDtypeStruct((K, L), jnp.float32),
            jax.ShapeDtypeStruct((K, L), jnp.float32),
            jax.ShapeDtypeStruct((1, L), jnp.float32),
        ],
    )(u, w.reshape(K, 1), W_mu, b_mu.reshape(1, L), W_lv, b_lv.reshape(1, L))
    s = _sc_accumulate(z, c0, c1, c2)
    out = pl.pallas_call(
        _finish_body,
        out_shape=jax.ShapeDtypeStruct((B, L), jnp.float32),
    )(s, m_l)
    return out


# speedup vs baseline: 2.1297x; 2.1297x over previous
import functools

import jax
import jax.numpy as jnp
from jax import lax
from jax.experimental import pallas as pl
from jax.experimental.pallas import tpu as pltpu
from jax.experimental.pallas import tpu_sc as plsc

_L = 64
_HALF_LOG_2PI = 0.9189385332046727
_LOG2E = 1.4426950408889634
_KC = 128
_NW = 32
_LANES = 16
_KB = 8
_RB = 128

_BT = 768


def _prep_body(u_ref, w_ref, Wmu_ref, bmu_ref, Wlv_ref, blv_ref,
               c0_ref, c1_ref, c2_ref, t0_ref, t1_ref, t2_ref, m_ref):
    mean = jnp.dot(u_ref[...], Wmu_ref[...],
                   preferred_element_type=jnp.float32) + bmu_ref[...]
    lv = jnp.dot(u_ref[...], Wlv_ref[...],
                 preferred_element_type=jnp.float32) + blv_ref[...]
    wv = w_ref[...]
    wmax = jnp.max(wv)
    logw = wv - (wmax + jnp.log(jnp.sum(jnp.exp(wv - wmax))))
    t = logw - 0.5 * lv - _HALF_LOG_2PI
    m_l = jnp.max(t, axis=0, keepdims=True)
    p = 0.5 * jnp.exp(-lv)
    c0 = (t - m_l) - p * mean * mean
    c1 = 2.0 * p * mean
    c2 = -p
    c0_ref[...] = c0
    c1_ref[...] = c1
    c2_ref[...] = c2
    t0_ref[...] = jnp.concatenate([c0, c0], axis=1) * _LOG2E
    t1_ref[...] = jnp.concatenate([c1, c1], axis=1) * _LOG2E
    t2_ref[...] = jnp.concatenate([c2, c2], axis=1) * _LOG2E
    m_ref[...] = m_l


def _tc_acc_body(z2_ref, t0_ref, t1_ref, t2_ref, s_ref):
    K = t0_ref.shape[0]
    nrows = z2_ref.shape[0]
    for c in range(nrows // _RB):
        z = z2_ref[c * _RB:(c + 1) * _RB, :]

        def body(i, s):
            base = i * _KB
            r0 = t0_ref[pl.ds(base, _KB), :]
            r1 = t1_ref[pl.ds(base, _KB), :]
            r2 = t2_ref[pl.ds(base, _KB), :]
            for j in range(_KB):
                t = r2[j:j + 1, :] * z + r1[j:j + 1, :]
                arg = t * z + r0[j:j + 1, :]
                s = s + jnp.exp2(arg)
            return s

        s = lax.fori_loop(0, K // _KB, body,
                          jnp.zeros((_RB, 2 * _L), jnp.float32))
        s_ref[c * _RB:(c + 1) * _RB, :] = s


def _finish_body(s_ref, m_ref, out_ref):
    out_ref[...] = m_ref[...] + jnp.log(s_ref[...])


def _sc_accumulate(z_tail, c0, c1, c2):
    Bs, L = z_tail.shape
    K = c0.shape[0]
    rows = Bs // _NW
    gr = min(_LANES, rows)
    mesh = plsc.VectorSubcoreMesh(core_axis_name="c", subcore_axis_name="s")

    @functools.partial(
        pl.kernel, mesh=mesh,
        out_type=jax.ShapeDtypeStruct((Bs, L), jnp.float32),
        scratch_types=[
            pltpu.VMEM((rows, L), jnp.float32),
            pltpu.VMEM((rows, L), jnp.float32),
            pltpu.VMEM((_KC, L), jnp.float32),
            pltpu.VMEM((_KC, L), jnp.float32),
            pltpu.VMEM((_KC, L), jnp.float32),
        ],
    )
    def sc_kernel(z_hbm, c0_hbm, c1_hbm, c2_hbm, s_hbm,
                  zv, sv, b0, b1, b2):
        wid = lax.axis_index("s") * 2 + lax.axis_index("c")
        base = wid * rows
        pltpu.sync_copy(z_hbm.at[pl.ds(base, rows)], zv)
        nchunks = K // _KC
        for c in range(nchunks):
            pltpu.sync_copy(c0_hbm.at[pl.ds(c * _KC, _KC)], b0)
            pltpu.sync_copy(c1_hbm.at[pl.ds(c * _KC, _KC)], b1)
            pltpu.sync_copy(c2_hbm.at[pl.ds(c * _KC, _KC)], b2)
            for g in range(L // _LANES):
                sl = pl.ds(g * _LANES, _LANES)
                for rr in range(rows // gr):
                    zr = [zv[rr * gr + r, sl] for r in range(gr)]

                    def body(k, acc):
                        v0 = b0[k, sl]
                        v1 = b1[k, sl]
                        v2 = b2[k, sl]
                        return tuple(
                            acc[r] + jnp.exp((v2 * zr[r] + v1) * zr[r] + v0)
                            for r in range(gr))

                    acc = lax.fori_loop(
                        0, _KC, body,
                        tuple(jnp.zeros((_LANES,), jnp.float32)
                              for _ in range(gr)))
                    for r in range(gr):
                        row = rr * gr + r
                        if c == 0:
                            sv[row, sl] = acc[r]
                        else:
                            sv[row, sl] = sv[row, sl] + acc[r]
        pltpu.sync_copy(sv, s_hbm.at[pl.ds(base, rows)])

    return sc_kernel(z_tail, c0, c1, c2)


@jax.jit
def kernel(z, u, w, W_mu, b_mu, W_lv, b_lv):
    B, L = z.shape
    K = u.shape[0]
    c0, c1, c2, t0, t1, t2, m_l = pl.pallas_call(
        _prep_body,
        out_shape=[
            jax.ShapeDtypeStruct((K, L), jnp.float32),
            jax.ShapeDtypeStruct((K, L), jnp.float32),
            jax.ShapeDtypeStruct((K, L), jnp.float32),
            jax.ShapeDtypeStruct((K, 2 * L), jnp.float32),
            jax.ShapeDtypeStruct((K, 2 * L), jnp.float32),
            jax.ShapeDtypeStruct((K, 2 * L), jnp.float32),
            jax.ShapeDtypeStruct((1, L), jnp.float32),
        ],
    )(u, w.reshape(K, 1), W_mu, b_mu.reshape(1, L), W_lv, b_lv.reshape(1, L))
    z2 = z[:_BT].reshape(_BT // 2, 2 * L)
    s_tc2 = pl.pallas_call(
        _tc_acc_body,
        out_shape=jax.ShapeDtypeStruct((_BT // 2, 2 * L), jnp.float32),
    )(z2, t0, t1, t2)
    s_sc = _sc_accumulate(z[_BT:], c0, c1, c2)
    s = jnp.concatenate([s_tc2.reshape(_BT, L), s_sc], axis=0)
    out = pl.pallas_call(
        _finish_body,
        out_shape=jax.ShapeDtypeStruct((B, L), jnp.float32),
    )(s, m_l)
    return out


# speedup vs baseline: 2.1838x; 1.0254x over previous
import functools

import jax
import jax.numpy as jnp
from jax import lax
from jax.experimental import pallas as pl
from jax.experimental.pallas import tpu as pltpu
from jax.experimental.pallas import tpu_sc as plsc

_L = 64
_HALF_LOG_2PI = 0.9189385332046727
_LOG2E = 1.4426950408889634
_KC = 256
_KU = 4
_NW = 32
_LANES = 16
_KB = 32
_RB = 128

_BT = 768


def _prep_body(u_ref, w_ref, Wmu_ref, bmu_ref, Wlv_ref, blv_ref,
               c0_ref, c1_ref, c2_ref, t0_ref, t1_ref, t2_ref, m_ref):
    mean = jnp.dot(u_ref[...], Wmu_ref[...],
                   preferred_element_type=jnp.float32) + bmu_ref[...]
    lv = jnp.dot(u_ref[...], Wlv_ref[...],
                 preferred_element_type=jnp.float32) + blv_ref[...]
    wv = w_ref[...]
    wmax = jnp.max(wv)
    logw = wv - (wmax + jnp.log(jnp.sum(jnp.exp(wv - wmax))))
    t = logw - 0.5 * lv - _HALF_LOG_2PI
    m_l = jnp.max(t, axis=0, keepdims=True)
    p = 0.5 * jnp.exp(-lv)
    c0 = (t - m_l) - p * mean * mean
    c1 = 2.0 * p * mean
    c2 = -p
    c0_ref[...] = c0
    c1_ref[...] = c1
    c2_ref[...] = c2
    t0_ref[...] = jnp.concatenate([c0, c0], axis=1) * _LOG2E
    t1_ref[...] = jnp.concatenate([c1, c1], axis=1) * _LOG2E
    t2_ref[...] = jnp.concatenate([c2, c2], axis=1) * _LOG2E
    m_ref[...] = m_l


def _tc_acc_body(z2_ref, t0_ref, t1_ref, t2_ref, s_ref):
    K = t0_ref.shape[0]
    nrows = z2_ref.shape[0]
    for c in range(nrows // _RB):
        z = z2_ref[c * _RB:(c + 1) * _RB, :]

        def body(i, s):
            base = i * _KB
            r0 = t0_ref[pl.ds(base, _KB), :]
            r1 = t1_ref[pl.ds(base, _KB), :]
            r2 = t2_ref[pl.ds(base, _KB), :]
            for j in range(_KB):
                t = r2[j:j + 1, :] * z + r1[j:j + 1, :]
                arg = t * z + r0[j:j + 1, :]
                s = s + jnp.exp2(arg)
            return s

        s = lax.fori_loop(0, K // _KB, body,
                          jnp.zeros((_RB, 2 * _L), jnp.float32))
        s_ref[c * _RB:(c + 1) * _RB, :] = s


def _finish_body(s_ref, m_ref, out_ref):
    out_ref[...] = m_ref[...] + jnp.log(s_ref[...])


def _sc_accumulate(z_tail, c0, c1, c2):
    Bs, L = z_tail.shape
    K = c0.shape[0]
    rows = Bs // _NW
    gr = min(_LANES, rows)
    mesh = plsc.VectorSubcoreMesh(core_axis_name="c", subcore_axis_name="s")

    @functools.partial(
        pl.kernel, mesh=mesh,
        out_type=jax.ShapeDtypeStruct((Bs, L), jnp.float32),
        scratch_types=[
            pltpu.VMEM((rows, L), jnp.float32),
            pltpu.VMEM((rows, L), jnp.float32),
            pltpu.VMEM((_KC, L), jnp.float32),
            pltpu.VMEM((_KC, L), jnp.float32),
            pltpu.VMEM((_KC, L), jnp.float32),
        ],
    )
    def sc_kernel(z_hbm, c0_hbm, c1_hbm, c2_hbm, s_hbm,
                  zv, sv, b0, b1, b2):
        wid = lax.axis_index("s") * 2 + lax.axis_index("c")
        base = wid * rows
        pltpu.sync_copy(z_hbm.at[pl.ds(base, rows)], zv)
        nchunks = K // _KC
        for c in range(nchunks):
            pltpu.sync_copy(c0_hbm.at[pl.ds(c * _KC, _KC)], b0)
            pltpu.sync_copy(c1_hbm.at[pl.ds(c * _KC, _KC)], b1)
            pltpu.sync_copy(c2_hbm.at[pl.ds(c * _KC, _KC)], b2)
            for g in range(L // _LANES):
                sl = pl.ds(g * _LANES, _LANES)
                for rr in range(rows // gr):
                    zr = [zv[rr * gr + r, sl] for r in range(gr)]

                    def body(i, acc):
                        for jj in range(_KU):
                            k = i * _KU + jj
                            v0 = b0[k, sl]
                            v1 = b1[k, sl]
                            v2 = b2[k, sl]
                            acc = tuple(
                                acc[r]
                                + jnp.exp((v2 * zr[r] + v1) * zr[r] + v0)
                                for r in range(gr))
                        return acc

                    acc = lax.fori_loop(
                        0, _KC // _KU, body,
                        tuple(jnp.zeros((_LANES,), jnp.float32)
                              for _ in range(gr)))
                    for r in range(gr):
                        row = rr * gr + r
                        if c == 0:
                            sv[row, sl] = acc[r]
                        else:
                            sv[row, sl] = sv[row, sl] + acc[r]
        pltpu.sync_copy(sv, s_hbm.at[pl.ds(base, rows)])

    return sc_kernel(z_tail, c0, c1, c2)


@jax.jit
def kernel(z, u, w, W_mu, b_mu, W_lv, b_lv):
    B, L = z.shape
    K = u.shape[0]
    c0, c1, c2, t0, t1, t2, m_l = pl.pallas_call(
        _prep_body,
        out_shape=[
            jax.ShapeDtypeStruct((K, L), jnp.float32),
            jax.ShapeDtypeStruct((K, L), jnp.float32),
            jax.ShapeDtypeStruct((K, L), jnp.float32),
            jax.ShapeDtypeStruct((K, 2 * L), jnp.float32),
            jax.ShapeDtypeStruct((K, 2 * L), jnp.float32),
            jax.ShapeDtypeStruct((K, 2 * L), jnp.float32),
            jax.ShapeDtypeStruct((1, L), jnp.float32),
        ],
    )(u, w.reshape(K, 1), W_mu, b_mu.reshape(1, L), W_lv, b_lv.reshape(1, L))
    s_sc = _sc_accumulate(z[_BT:], c0, c1, c2)
    z2 = z[:_BT].reshape(_BT // 2, 2 * L)
    s_tc2 = pl.pallas_call(
        _tc_acc_body,
        out_shape=jax.ShapeDtypeStruct((_BT // 2, 2 * L), jnp.float32),
    )(z2, t0, t1, t2)
    s = jnp.concatenate([s_tc2.reshape(_BT, L), s_sc], axis=0)
    out = pl.pallas_call(
        _finish_body,
        out_shape=jax.ShapeDtypeStruct((B, L), jnp.float32),
    )(s, m_l)
    return out


# speedup vs baseline: 3.8974x; 1.7847x over previous
import jax
import jax.numpy as jnp
from jax.experimental import pallas as pl
from jax.experimental.pallas import tpu as pltpu

_L = 64
_HALF_LOG_2PI = 0.9189385332046727
_KB = 64
_LOG2E = 1.4426950408889634
_RB = 128


def _tc_body(z2_ref, u_ref, w_ref, Wmu_ref, bmu_ref, Wlv_ref, blv_ref,
             out_ref, c0_ref, c1_ref, c2_ref):
    K = u_ref.shape[0]
    mean = jnp.dot(u_ref[...], Wmu_ref[...],
                   preferred_element_type=jnp.float32) + bmu_ref[...]
    lv = jnp.dot(u_ref[...], Wlv_ref[...],
                 preferred_element_type=jnp.float32) + blv_ref[...]
    wv = w_ref[...]
    wmax = jnp.max(wv)
    logw = wv - (wmax + jnp.log(jnp.sum(jnp.exp(wv - wmax))))
    t = logw - 0.5 * lv - _HALF_LOG_2PI
    m_l = jnp.max(t, axis=0, keepdims=True)
    p = 0.5 * jnp.exp(-lv)
    c0 = ((t - m_l) - p * mean * mean) * _LOG2E
    c1 = (2.0 * _LOG2E) * p * mean
    c2 = (-_LOG2E) * p
    c0_ref[...] = jnp.concatenate([c0, c0], axis=1)
    c1_ref[...] = jnp.concatenate([c1, c1], axis=1)
    c2_ref[...] = jnp.concatenate([c2, c2], axis=1)

    m_t = jnp.concatenate([m_l, m_l], axis=1)
    nrows = z2_ref.shape[0]
    for c in range(nrows // _RB):
        z = z2_ref[c * _RB:(c + 1) * _RB, :]

        def body(i, s):
            base = i * _KB
            r0 = c0_ref[pl.ds(base, _KB), :]
            r1 = c1_ref[pl.ds(base, _KB), :]
            r2 = c2_ref[pl.ds(base, _KB), :]
            for j in range(_KB):
                t = r2[j:j + 1, :] * z + r1[j:j + 1, :]
                arg = t * z + r0[j:j + 1, :]
                s = s + jnp.exp2(arg)
            return s

        s = jax.lax.fori_loop(0, K // _KB, body,
                              jnp.zeros((_RB, 2 * _L), jnp.float32))
        out_ref[c * _RB:(c + 1) * _RB, :] = m_t + jnp.log(s)


@jax.jit
def kernel(z, u, w, W_mu, b_mu, W_lv, b_lv):
    B, L = z.shape
    K = u.shape[0]
    z2 = z.reshape(B // 2, 2 * L)
    out2 = pl.pallas_call(
        _tc_body,
        out_shape=jax.ShapeDtypeStruct((B // 2, 2 * L), jnp.float32),
        scratch_shapes=[
            pltpu.VMEM((K, 2 * L), jnp.float32),
            pltpu.VMEM((K, 2 * L), jnp.float32),
            pltpu.VMEM((K, 2 * L), jnp.float32),
        ],
    )(z2, u, w.reshape(K, 1), W_mu, b_mu.reshape(1, L), W_lv,
      b_lv.reshape(1, L))
    return out2.reshape(B, L)
